# baseline (speedup 1.0000x reference)
import functools

import jax
import jax.numpy as jnp
from jax import lax
from jax.experimental import pallas as pl
from jax.experimental.pallas import tpu as pltpu
from jax.experimental.pallas import tpu_sc as plsc

_EPS = 1e-30
_LN2 = 0.6931471805599453
_B = 16384
_V = 29
_T = 32
_L = 16


def _log16(x):
    bits = lax.bitcast_convert_type(x, jnp.int32)
    e = (bits >> 23) - 127
    m = lax.bitcast_convert_type((bits & 0x007FFFFF) | 0x3F800000, jnp.float32)
    big = m > 1.4142135381698608
    m = jnp.where(big, m * 0.5, m)
    ef = e.astype(jnp.float32) + jnp.where(big, 1.0, 0.0)
    s = (m - 1.0) / (m + 1.0)
    z = s * s
    poly = 2.0 + z * (0.6666666666 + z * (0.4 + z * 0.2857142857))
    return ef * _LN2 + s * poly


def _make_sc_kernel():
    info = plsc.get_sparse_core_info()
    nc, ns = info.num_cores, info.num_subcores
    nw = nc * ns
    bpw = _B // nw
    mesh = plsc.VectorSubcoreMesh(core_axis_name="c", subcore_axis_name="s")

    @functools.partial(
        pl.kernel,
        mesh=mesh,
        out_type=jax.ShapeDtypeStruct((_B,), jnp.float32),
        compiler_params=pltpu.CompilerParams(needs_layout_passes=False),
        scratch_types=[
            pltpu.VMEM((_T,), jnp.float32),
            pltpu.VMEM((bpw,), jnp.int32),
            pltpu.VMEM((bpw,), jnp.float32),
            pltpu.SemaphoreType.DMA,
            pltpu.SemaphoreType.DMA,
        ],
    )
    def sc_kernel(prob_hbm, idx_hbm, out_hbm, tab_v, idx_v, out_v, sem, osem):
        wid = lax.axis_index("s") * nc + lax.axis_index("c")
        base = wid * bpw
        idx_cp = pltpu.async_copy(idx_hbm.at[pl.ds(base, bpw)], idx_v, sem)
        pltpu.sync_copy(prob_hbm, tab_v.at[pl.ds(0, _V)])
        for j in range(_T // _L):
            x = tab_v[pl.ds(j * _L, _L)] + _EPS
            tab_v[pl.ds(j * _L, _L)] = _log16(x)
        idx_cp.wait()
        nchunk = 4
        chunk = bpw // nchunk
        out_cps = []
        for h in range(nchunk):
            for i in range(h * chunk // _L, (h + 1) * chunk // _L):
                idx = idx_v[pl.ds(i * _L, _L)]
                out_v[pl.ds(i * _L, _L)] = plsc.load_gather(tab_v, [idx])
            out_cps.append(pltpu.async_copy(
                out_v.at[pl.ds(h * chunk, chunk)],
                out_hbm.at[pl.ds(base + h * chunk, chunk)], osem))
        for cp in out_cps:
            cp.wait()

    return sc_kernel


_SC_KERNEL = _make_sc_kernel()


def kernel(batch_n_nodes, prob):
    return _SC_KERNEL(prob.astype(jnp.float32), batch_n_nodes.astype(jnp.int32))


# speedup vs baseline: 1.0015x; 1.0015x over previous
import functools

import jax
import jax.numpy as jnp
from jax import lax
from jax.experimental import pallas as pl
from jax.experimental.pallas import tpu as pltpu
from jax.experimental.pallas import tpu_sc as plsc

_EPS = 1e-30
_LN2 = 0.6931471805599453
_B = 16384
_V = 29
_T = 32
_L = 16


def _log16(x):
    bits = lax.bitcast_convert_type(x, jnp.int32)
    e = (bits >> 23) - 127
    m = lax.bitcast_convert_type((bits & 0x007FFFFF) | 0x3F800000, jnp.float32)
    big = m > 1.4142135381698608
    m = jnp.where(big, m * 0.5, m)
    ef = e.astype(jnp.float32) + jnp.where(big, 1.0, 0.0)
    s = (m - 1.0) / (m + 1.0)
    z = s * s
    poly = 2.0 + z * (0.6666666666 + z * (0.4 + z * 0.2857142857))
    return ef * _LN2 + s * poly


def _make_sc_kernel():
    info = plsc.get_sparse_core_info()
    nc, ns = info.num_cores, info.num_subcores
    nw = nc * ns
    bpw = _B // nw
    mesh = plsc.VectorSubcoreMesh(core_axis_name="c", subcore_axis_name="s")

    @functools.partial(
        pl.kernel,
        mesh=mesh,
        out_type=jax.ShapeDtypeStruct((_B,), jnp.float32),
        compiler_params=pltpu.CompilerParams(needs_layout_passes=False),
        scratch_types=[
            pltpu.VMEM((_T,), jnp.float32),
            pltpu.VMEM((bpw,), jnp.int32),
            pltpu.VMEM((bpw,), jnp.float32),
            pltpu.SemaphoreType.DMA,
            pltpu.SemaphoreType.DMA,
            pltpu.SemaphoreType.DMA,
        ],
    )
    def sc_kernel(prob_hbm, idx_hbm, out_hbm, tab_v, idx_v, out_v,
                  sem0, sem1, osem):
        wid = lax.axis_index("s") * nc + lax.axis_index("c")
        base = wid * bpw
        half = bpw // 2
        idx_cps = [
            pltpu.async_copy(idx_hbm.at[pl.ds(base + h * half, half)],
                             idx_v.at[pl.ds(h * half, half)], isem)
            for h, isem in ((0, sem0), (1, sem1))
        ]
        pltpu.sync_copy(prob_hbm, tab_v.at[pl.ds(0, _V)])
        for j in range(_T // _L):
            x = tab_v[pl.ds(j * _L, _L)] + _EPS
            tab_v[pl.ds(j * _L, _L)] = _log16(x)
        out_cps = []
        for h in range(2):
            idx_cps[h].wait()
            for i in range(h * half // _L, (h + 1) * half // _L):
                idx = idx_v[pl.ds(i * _L, _L)]
                out_v[pl.ds(i * _L, _L)] = plsc.load_gather(tab_v, [idx])
            out_cps.append(pltpu.async_copy(
                out_v.at[pl.ds(h * half, half)],
                out_hbm.at[pl.ds(base + h * half, half)], osem))
        for cp in out_cps:
            cp.wait()

    return sc_kernel


_SC_KERNEL = _make_sc_kernel()


def kernel(batch_n_nodes, prob):
    return _SC_KERNEL(prob.astype(jnp.float32), batch_n_nodes.astype(jnp.int32))


# speedup vs baseline: 1.0056x; 1.0041x over previous
import functools

import jax
import jax.numpy as jnp
from jax import lax
from jax.experimental import pallas as pl
from jax.experimental.pallas import tpu as pltpu
from jax.experimental.pallas import tpu_sc as plsc

_EPS = 1e-30
_LN2 = 0.6931471805599453
_B = 16384
_V = 29
_T = 32
_L = 16

_C = (9.975032552119053e-06, 0.9992354838332771, -0.4902307234234269,
      0.28527268109062165, -0.13158182508881333, 0.03044900453868939)


def _log16(x):
    bits = lax.bitcast_convert_type(x, jnp.int32)
    e = (bits >> 23) - 127
    t = lax.bitcast_convert_type(
        (bits & 0x007FFFFF) | 0x3F800000, jnp.float32) - 1.0
    p = _C[0] + t * (_C[1] + t * (_C[2] + t * (_C[3] + t * (_C[4] + t * _C[5]))))
    return e.astype(jnp.float32) * _LN2 + p


def _make_sc_kernel():
    info = plsc.get_sparse_core_info()
    nc, ns = info.num_cores, info.num_subcores
    nw = nc * ns
    bpw = _B // nw
    mesh = plsc.VectorSubcoreMesh(core_axis_name="c", subcore_axis_name="s")

    @functools.partial(
        pl.kernel,
        mesh=mesh,
        out_type=jax.ShapeDtypeStruct((_B,), jnp.float32),
        compiler_params=pltpu.CompilerParams(needs_layout_passes=False),
        scratch_types=[
            pltpu.VMEM((_T,), jnp.float32),
            pltpu.VMEM((bpw,), jnp.int32),
            pltpu.VMEM((bpw,), jnp.float32),
            pltpu.SemaphoreType.DMA,
            pltpu.SemaphoreType.DMA,
        ],
    )
    def sc_kernel(prob_hbm, idx_hbm, out_hbm, tab_v, idx_v, out_v, sem, osem):
        wid = lax.axis_index("s") * nc + lax.axis_index("c")
        base = wid * bpw
        idx_cp = pltpu.async_copy(idx_hbm.at[pl.ds(base, bpw)], idx_v, sem)
        pltpu.sync_copy(prob_hbm, tab_v.at[pl.ds(0, _V)])
        for j in range(_T // _L):
            x = tab_v[pl.ds(j * _L, _L)] + _EPS
            tab_v[pl.ds(j * _L, _L)] = _log16(x)
        idx_cp.wait()
        half = bpw // 2
        out_cps = []
        for h in range(2):
            for i in range(h * half // _L, (h + 1) * half // _L):
                idx = idx_v[pl.ds(i * _L, _L)]
                out_v[pl.ds(i * _L, _L)] = plsc.load_gather(tab_v, [idx])
            out_cps.append(pltpu.async_copy(
                out_v.at[pl.ds(h * half, half)],
                out_hbm.at[pl.ds(base + h * half, half)], osem))
        for cp in out_cps:
            cp.wait()

    return sc_kernel


_SC_KERNEL = _make_sc_kernel()


def kernel(batch_n_nodes, prob):
    return _SC_KERNEL(prob.astype(jnp.float32), batch_n_nodes.astype(jnp.int32))


# speedup vs baseline: 1.0581x; 1.0522x over previous
import functools

import jax
import jax.numpy as jnp
from jax import lax
from jax.experimental import pallas as pl
from jax.experimental.pallas import tpu as pltpu
from jax.experimental.pallas import tpu_sc as plsc

_EPS = 1e-30
_LN2 = 0.6931471805599453
_B = 16384
_V = 29
_T = 32
_L = 16

_C = (9.975032552119053e-06, 0.9992354838332771, -0.4902307234234269,
      0.28527268109062165, -0.13158182508881333, 0.03044900453868939)


def _log16(x):
    bits = lax.bitcast_convert_type(x, jnp.int32)
    e = (bits >> 23) - 127
    t = lax.bitcast_convert_type(
        (bits & 0x007FFFFF) | 0x3F800000, jnp.float32) - 1.0
    p = _C[0] + t * (_C[1] + t * (_C[2] + t * (_C[3] + t * (_C[4] + t * _C[5]))))
    return e.astype(jnp.float32) * _LN2 + p


def _make_sc_kernel():
    info = plsc.get_sparse_core_info()
    nc, ns = 1, info.num_subcores
    nw = nc * ns
    bpw = _B // nw
    mesh = plsc.VectorSubcoreMesh(core_axis_name="c", subcore_axis_name="s",
                                  num_cores=1)

    @functools.partial(
        pl.kernel,
        mesh=mesh,
        out_type=jax.ShapeDtypeStruct((_B,), jnp.float32),
        compiler_params=pltpu.CompilerParams(needs_layout_passes=False),
        scratch_types=[
            pltpu.VMEM((_T,), jnp.float32),
            pltpu.VMEM((bpw,), jnp.int32),
            pltpu.VMEM((bpw,), jnp.float32),
            pltpu.SemaphoreType.DMA,
            pltpu.SemaphoreType.DMA,
        ],
    )
    def sc_kernel(prob_hbm, idx_hbm, out_hbm, tab_v, idx_v, out_v, sem, osem):
        wid = lax.axis_index("s") * nc + lax.axis_index("c")
        base = wid * bpw
        idx_cp = pltpu.async_copy(idx_hbm.at[pl.ds(base, bpw)], idx_v, sem)
        pltpu.sync_copy(prob_hbm, tab_v.at[pl.ds(0, _V)])
        for j in range(_T // _L):
            x = tab_v[pl.ds(j * _L, _L)] + _EPS
            tab_v[pl.ds(j * _L, _L)] = _log16(x)
        idx_cp.wait()
        half = bpw // 2
        out_cps = []
        for h in range(2):
            for i in range(h * half // _L, (h + 1) * half // _L):
                idx = idx_v[pl.ds(i * _L, _L)]
                out_v[pl.ds(i * _L, _L)] = plsc.load_gather(tab_v, [idx])
            out_cps.append(pltpu.async_copy(
                out_v.at[pl.ds(h * half, half)],
                out_hbm.at[pl.ds(base + h * half, half)], osem))
        for cp in out_cps:
            cp.wait()

    return sc_kernel


_SC_KERNEL = _make_sc_kernel()


def kernel(batch_n_nodes, prob):
    return _SC_KERNEL(prob.astype(jnp.float32), batch_n_nodes.astype(jnp.int32))


# speedup vs baseline: 1.0676x; 1.0090x over previous
import functools

import jax
import jax.numpy as jnp
from jax import lax
from jax.experimental import pallas as pl
from jax.experimental.pallas import tpu as pltpu
from jax.experimental.pallas import tpu_sc as plsc

_EPS = 1e-30
_LN2 = 0.6931471805599453
_B = 16384
_V = 29
_T = 32
_L = 16

_C = (9.975032552119053e-06, 0.9992354838332771, -0.4902307234234269,
      0.28527268109062165, -0.13158182508881333, 0.03044900453868939)


def _log16(x):
    bits = lax.bitcast_convert_type(x, jnp.int32)
    e = (bits >> 23) - 127
    t = lax.bitcast_convert_type(
        (bits & 0x007FFFFF) | 0x3F800000, jnp.float32) - 1.0
    p = _C[0] + t * (_C[1] + t * (_C[2] + t * (_C[3] + t * (_C[4] + t * _C[5]))))
    return e.astype(jnp.float32) * _LN2 + p


def _make_sc_kernel():
    info = plsc.get_sparse_core_info()
    nc, ns = 1, info.num_subcores
    nw = nc * ns
    bpw = _B // nw
    mesh = plsc.VectorSubcoreMesh(core_axis_name="c", subcore_axis_name="s",
                                  num_cores=1)

    @functools.partial(
        pl.kernel,
        mesh=mesh,
        out_type=jax.ShapeDtypeStruct((_B,), jnp.float32),
        compiler_params=pltpu.CompilerParams(needs_layout_passes=False),
        scratch_types=[
            pltpu.VMEM((_T,), jnp.float32),
            pltpu.VMEM((bpw,), jnp.int32),
            pltpu.VMEM((bpw,), jnp.float32),
            pltpu.SemaphoreType.DMA,
            pltpu.SemaphoreType.DMA,
        ],
    )
    def sc_kernel(prob_hbm, idx_hbm, out_hbm, tab_v, idx_v, out_v, sem, osem):
        wid = lax.axis_index("s") * nc + lax.axis_index("c")
        base = wid * bpw
        idx_cp = pltpu.async_copy(idx_hbm.at[pl.ds(base, bpw)], idx_v, sem)
        pltpu.sync_copy(prob_hbm, tab_v.at[pl.ds(0, _V)])
        for j in range(_T // _L):
            x = tab_v[pl.ds(j * _L, _L)] + _EPS
            tab_v[pl.ds(j * _L, _L)] = _log16(x)
        idx_cp.wait()
        nchunk = 4
        chunk = bpw // nchunk
        out_cps = []
        for h in range(nchunk):
            for i in range(h * chunk // _L, (h + 1) * chunk // _L):
                idx = idx_v[pl.ds(i * _L, _L)]
                out_v[pl.ds(i * _L, _L)] = plsc.load_gather(tab_v, [idx])
            out_cps.append(pltpu.async_copy(
                out_v.at[pl.ds(h * chunk, chunk)],
                out_hbm.at[pl.ds(base + h * chunk, chunk)], osem))
        for cp in out_cps:
            cp.wait()

    return sc_kernel


_SC_KERNEL = _make_sc_kernel()


def kernel(batch_n_nodes, prob):
    return _SC_KERNEL(prob.astype(jnp.float32), batch_n_nodes.astype(jnp.int32))
